# Initial kernel scaffold; baseline (speedup 1.0000x reference)
import jax
import jax.numpy as jnp
from jax.experimental import pallas as pl


def kernel(xyz, features, W1, b1, W2, b2, Wa, ba):
    raise NotImplementedError("write your pallas kernel here")


# speedup vs baseline: 73.0923x; 73.0923x over previous
import functools

import jax
import jax.numpy as jnp
from jax import lax
from jax.experimental import pallas as pl
from jax.experimental.pallas import tpu as pltpu
from jax.experimental.pallas import tpu_sc as plsc

_B = 4
_K = 16384
_FEAT = 256
_TOKEN_DIM = 256
_T = 1024
_RPD = 3
_NREG = 27
_KT = 2048
_R = 128

_NC = 2
_NS = 16
_NW = _NC * _NS
_BPW = (_B * _T) // _NW


def _imp_body(f_ref, w1_ref, b1_ref, w2_ref, b2_ref, o_ref):
    f = f_ref[0]
    h = lax.dot_general(w1_ref[...], f, (((0,), (0,)), ((), ())),
                        preferred_element_type=jnp.float32)
    h = jnp.maximum(h + b1_ref[...], 0.0)
    y = lax.dot_general(w2_ref[...], h, (((0,), (0,)), ((), ())),
                        preferred_element_type=jnp.float32)
    y = y + b2_ref[0, 0]
    o_ref[0, 0] = jnp.maximum(y, 0.0) + jnp.log1p(jnp.exp(-jnp.abs(y)))


def _importance(features, W1, b1, W2, b2):
    nt = _K // _KT
    return pl.pallas_call(
        _imp_body,
        grid=(_B, nt),
        in_specs=[
            pl.BlockSpec((1, _FEAT, _KT), lambda b, t: (b, 0, t)),
            pl.BlockSpec((_FEAT, _FEAT // 2), lambda b, t: (0, 0)),
            pl.BlockSpec((_FEAT // 2, 1), lambda b, t: (0, 0)),
            pl.BlockSpec((_FEAT // 2, 1), lambda b, t: (0, 0)),
            pl.BlockSpec(memory_space=pltpu.SMEM),
        ],
        out_specs=pl.BlockSpec((1, 1, 1, _KT), lambda b, t: (b, t, 0, 0)),
        out_shape=jax.ShapeDtypeStruct((_B, nt, 1, _KT), jnp.float32),
    )(features, W1, b1.reshape(_FEAT // 2, 1), W2, b2.reshape(1, 1))


def _region_and_valid(xs, ys, zs):
    mnx, mxx = jnp.min(xs), jnp.max(xs)
    mny, mxy = jnp.min(ys), jnp.max(ys)
    mnz, mxz = jnp.min(zs), jnp.max(zs)
    xn = (xs - mnx) / (mxx - mnx + 1e-06)
    yn = (ys - mny) / (mxy - mny + 1e-06)
    zn = (zs - mnz) / (mxz - mnz + 1e-06)
    rix = jnp.clip(xn * _RPD, 0, _RPD - 1).astype(jnp.int32)
    riy = jnp.clip(yn * _RPD, 0, _RPD - 1).astype(jnp.int32)
    riz = jnp.clip(zn * _RPD, 0, _RPD - 1).astype(jnp.int32)
    region = rix * (_RPD * _RPD) + riy * _RPD + riz
    valid = ((jnp.abs(xs) + jnp.abs(ys)) + jnp.abs(zs)) > 0
    return region, valid


def _stats_body(xyz_ref, imp_ref, rimp_ref, cnt_ref):
    b = pl.program_id(0)
    xs, ys, zs = xyz_ref[0, 0], xyz_ref[0, 1], xyz_ref[0, 2]
    imp = imp_ref[0]
    region, valid = _region_and_valid(xs, ys, zs)
    for r in range(_NREG):
        m = (region == r) & valid
        rimp_ref[b, r] = jnp.sum(jnp.where(m, imp, 0.0))
        cnt_ref[b, r] = jnp.sum(m.astype(jnp.int32))


def _region_stats(xyzT, impf):
    return pl.pallas_call(
        _stats_body,
        grid=(_B,),
        in_specs=[
            pl.BlockSpec((1, 3, _R, _R), lambda b: (b, 0, 0, 0)),
            pl.BlockSpec((1, _R, _R), lambda b: (b, 0, 0)),
        ],
        out_specs=[
            pl.BlockSpec(memory_space=pltpu.SMEM),
            pl.BlockSpec(memory_space=pltpu.SMEM),
        ],
        out_shape=[
            jax.ShapeDtypeStruct((_B, _NREG), jnp.float32),
            jax.ShapeDtypeStruct((_B, _NREG), jnp.int32),
        ],
    )(xyzT, impf)


def _select_body(xyz_ref, take_ref, ufps_ref, sel_ref, xyzsel_ref):
    b = pl.program_id(0)
    xs, ys, zs = xyz_ref[0, 0], xyz_ref[0, 1], xyz_ref[0, 2]
    region, valid = _region_and_valid(xs, ys, zs)
    row = lax.broadcasted_iota(jnp.int32, (_R, _R), 0)
    col = lax.broadcasted_iota(jnp.int32, (_R, _R), 1)
    iota_i = row * _R + col
    iota_f = iota_i.astype(jnp.float32)
    neg = jnp.float32(-jnp.inf)

    def region_body(r, p):
        member = (region == r) & valid
        take = take_ref[b, r]
        fps_b = ufps_ref[b, r] > 0
        steps = jnp.minimum(take, _T - p)
        score0 = jnp.where(member,
                           jnp.where(fps_b, jnp.float32(1e10), -iota_f),
                           neg)

        def step(_, carry):
            p2, score = carry
            mx = jnp.max(score)
            far = jnp.min(jnp.where(score == mx, iota_i, jnp.int32(_K)))
            eq = iota_i == far
            cx = jnp.sum(jnp.where(eq, xs, 0.0))
            cy = jnp.sum(jnp.where(eq, ys, 0.0))
            cz = jnp.sum(jnp.where(eq, zs, 0.0))
            sel_ref[b, p2] = far
            xyzsel_ref[b, 0, p2] = cx
            xyzsel_ref[b, 1, p2] = cy
            xyzsel_ref[b, 2, p2] = cz
            dx = xs - cx
            dy = ys - cy
            dz = zs - cz
            d = (dx * dx + dy * dy) + dz * dz
            score = jnp.where(fps_b, jnp.minimum(score, d),
                              jnp.where(eq, neg, score))
            return (p2 + 1, score)

        p_out, _ = lax.fori_loop(0, steps, step, (p, score0))
        return p_out

    lax.fori_loop(0, _NREG, region_body, jnp.int32(0))


def _select(xyzT, take, ufps):
    return pl.pallas_call(
        _select_body,
        grid=(_B,),
        in_specs=[
            pl.BlockSpec((1, 3, _R, _R), lambda b: (b, 0, 0, 0)),
            pl.BlockSpec(memory_space=pltpu.SMEM),
            pl.BlockSpec(memory_space=pltpu.SMEM),
        ],
        out_specs=[
            pl.BlockSpec(memory_space=pltpu.SMEM),
            pl.BlockSpec(memory_space=pltpu.SMEM),
        ],
        out_shape=[
            jax.ShapeDtypeStruct((_B, _T), jnp.int32),
            jax.ShapeDtypeStruct((_B, 3, _T), jnp.float32),
        ],
    )(xyzT, take, ufps)


def _gather_rows(table, gidx):
    mesh = plsc.VectorSubcoreMesh(core_axis_name="c", subcore_axis_name="s")

    @functools.partial(
        pl.kernel,
        mesh=mesh,
        out_type=jax.ShapeDtypeStruct((_B * _T, _FEAT), jnp.float32),
        scratch_types=[
            pltpu.VMEM((_BPW,), jnp.int32),
            pltpu.VMEM((_BPW, _FEAT), jnp.float32),
            pltpu.SemaphoreType.DMA,
        ],
    )
    def k(table_hbm, idx_hbm, out_hbm, idx_v, rows_v, sem):
        wid = lax.axis_index("s") * _NC + lax.axis_index("c")
        base = wid * _BPW
        pltpu.sync_copy(idx_hbm.at[pl.ds(base, _BPW)], idx_v)
        pltpu.async_copy(table_hbm.at[idx_v], rows_v, sem).wait()
        pltpu.sync_copy(rows_v, out_hbm.at[pl.ds(base, _BPW)])

    return k(table, gidx)


def _proj_body(rows_ref, wa_ref, ba_ref, len_ref, o_ref):
    b = pl.program_id(0)
    rows = rows_ref[0]
    fo = lax.dot_general(wa_ref[...], rows, (((0,), (1,)), ((), ())),
                         preferred_element_type=jnp.float32)
    fo = fo + ba_ref[...]
    tpos = lax.broadcasted_iota(jnp.int32, (1, _T), 1)
    o_ref[0] = jnp.where(tpos < len_ref[b, 0], fo, 0.0)


def _project(rows, Wa, ba, sel_len):
    return pl.pallas_call(
        _proj_body,
        grid=(_B,),
        in_specs=[
            pl.BlockSpec((1, _T, _FEAT), lambda b: (b, 0, 0)),
            pl.BlockSpec((_FEAT, _TOKEN_DIM), lambda b: (0, 0)),
            pl.BlockSpec((_TOKEN_DIM, 1), lambda b: (0, 0)),
            pl.BlockSpec(memory_space=pltpu.SMEM),
        ],
        out_specs=pl.BlockSpec((1, _TOKEN_DIM, _T), lambda b: (b, 0, 0)),
        out_shape=jax.ShapeDtypeStruct((_B, _TOKEN_DIM, _T), jnp.float32),
    )(rows, Wa, ba.reshape(_TOKEN_DIM, 1), sel_len.reshape(_B, 1))


def kernel(xyz, features, W1, b1, W2, b2, Wa, ba):
    xyzT = jnp.transpose(xyz, (0, 2, 1)).reshape(_B, 3, _R, _R)

    imp = _importance(features, W1, b1, W2, b2)
    impf = imp.reshape(_B, _R, _R)

    rimp, cnt = _region_stats(xyzT, impf)

    n_rows, take_rows, len_rows = [], [], []
    for b in range(_B):
        total = rimp[b].sum() + 1e-08
        n_b = jnp.round(rimp[b] / total * _T).astype(jnp.int32)
        take_b = jnp.minimum(cnt[b], n_b)
        n_rows.append(n_b)
        take_rows.append(take_b)
        len_rows.append(jnp.minimum(jnp.sum(take_b), _T).astype(jnp.int32))
    n_r = jnp.stack(n_rows)
    take = jnp.stack(take_rows)
    sel_len = jnp.stack(len_rows)
    ufps = (cnt > n_r).astype(jnp.int32)

    sel, xyzsel = _select(xyzT, take, ufps)

    maskT = jnp.arange(_T)[None, :] < sel_len[:, None]
    xyz_out = jnp.where(maskT[:, :, None],
                        jnp.transpose(xyzsel, (0, 2, 1)), jnp.float32(0))
    sel_m = jnp.where(maskT, sel, 0)

    feat_rows = jnp.transpose(features, (0, 2, 1)).reshape(_B * _K, _FEAT)
    gidx = (sel_m + jnp.arange(_B, dtype=jnp.int32)[:, None] * _K)
    rows = _gather_rows(feat_rows, gidx.reshape(_B * _T))
    feat_out = _project(rows.reshape(_B, _T, _FEAT), Wa, ba, sel_len)
    return (xyz_out, feat_out)


# speedup vs baseline: 98.2457x; 1.3441x over previous
import functools

import jax
import jax.numpy as jnp
from jax import lax
from jax.experimental import pallas as pl
from jax.experimental.pallas import tpu as pltpu
from jax.experimental.pallas import tpu_sc as plsc

_B = 4
_K = 16384
_FEAT = 256
_TOKEN_DIM = 256
_T = 1024
_RPD = 3
_NREG = 27
_KT = 2048
_R = 128

_NC = 2
_NS = 16
_NW = _NC * _NS
_BPW = (_B * _T) // _NW


def _imp_body(f_ref, w1_ref, b1_ref, w2_ref, b2_ref, o_ref):
    f = f_ref[0]
    h = lax.dot_general(w1_ref[...], f, (((0,), (0,)), ((), ())),
                        preferred_element_type=jnp.float32)
    h = jnp.maximum(h + b1_ref[...], 0.0)
    y = lax.dot_general(w2_ref[...], h, (((0,), (0,)), ((), ())),
                        preferred_element_type=jnp.float32)
    y = y + b2_ref[0, 0]
    o_ref[0, 0] = jnp.maximum(y, 0.0) + jnp.log1p(jnp.exp(-jnp.abs(y)))


def _importance(features, W1, b1, W2, b2):
    nt = _K // _KT
    return pl.pallas_call(
        _imp_body,
        grid=(_B, nt),
        in_specs=[
            pl.BlockSpec((1, _FEAT, _KT), lambda b, t: (b, 0, t)),
            pl.BlockSpec((_FEAT, _FEAT // 2), lambda b, t: (0, 0)),
            pl.BlockSpec((_FEAT // 2, 1), lambda b, t: (0, 0)),
            pl.BlockSpec((_FEAT // 2, 1), lambda b, t: (0, 0)),
            pl.BlockSpec(memory_space=pltpu.SMEM),
        ],
        out_specs=pl.BlockSpec((1, 1, 1, _KT), lambda b, t: (b, t, 0, 0)),
        out_shape=jax.ShapeDtypeStruct((_B, nt, 1, _KT), jnp.float32),
    )(features, W1, b1.reshape(_FEAT // 2, 1), W2, b2.reshape(1, 1))


def _region_and_valid(xs, ys, zs):
    mnx, mxx = jnp.min(xs), jnp.max(xs)
    mny, mxy = jnp.min(ys), jnp.max(ys)
    mnz, mxz = jnp.min(zs), jnp.max(zs)
    xn = (xs - mnx) / (mxx - mnx + 1e-06)
    yn = (ys - mny) / (mxy - mny + 1e-06)
    zn = (zs - mnz) / (mxz - mnz + 1e-06)
    rix = jnp.clip(xn * _RPD, 0, _RPD - 1).astype(jnp.int32)
    riy = jnp.clip(yn * _RPD, 0, _RPD - 1).astype(jnp.int32)
    riz = jnp.clip(zn * _RPD, 0, _RPD - 1).astype(jnp.int32)
    region = rix * (_RPD * _RPD) + riy * _RPD + riz
    valid = ((jnp.abs(xs) + jnp.abs(ys)) + jnp.abs(zs)) > 0
    return region, valid


def _stats_body(xyz_ref, imp_ref, rimp_ref, cnt_ref):
    b = pl.program_id(0)
    xs, ys, zs = xyz_ref[0, 0], xyz_ref[0, 1], xyz_ref[0, 2]
    imp = imp_ref[0]
    region, valid = _region_and_valid(xs, ys, zs)
    for r in range(_NREG):
        m = (region == r) & valid
        rimp_ref[b, r] = jnp.sum(jnp.where(m, imp, 0.0))
        cnt_ref[b, r] = jnp.sum(m.astype(jnp.int32))


def _region_stats(xyzT, impf):
    return pl.pallas_call(
        _stats_body,
        grid=(_B,),
        in_specs=[
            pl.BlockSpec((1, 3, _R, _R), lambda b: (b, 0, 0, 0)),
            pl.BlockSpec((1, _R, _R), lambda b: (b, 0, 0)),
        ],
        out_specs=[
            pl.BlockSpec(memory_space=pltpu.SMEM),
            pl.BlockSpec(memory_space=pltpu.SMEM),
        ],
        out_shape=[
            jax.ShapeDtypeStruct((_B, _NREG), jnp.float32),
            jax.ShapeDtypeStruct((_B, _NREG), jnp.int32),
        ],
    )(xyzT, impf)


def _select_body(xyz_ref, xyzs_ref, take_ref, ufps_ref, sel_ref, xyzsel_ref):
    b = pl.program_id(0)
    xs, ys, zs = xyz_ref[0, 0], xyz_ref[0, 1], xyz_ref[0, 2]
    region, valid = _region_and_valid(xs, ys, zs)
    row = lax.broadcasted_iota(jnp.int32, (_R, _R), 0)
    col = lax.broadcasted_iota(jnp.int32, (_R, _R), 1)
    iota_i = row * _R + col
    iota_f = iota_i.astype(jnp.float32)
    neg = jnp.float32(-jnp.inf)

    def emit(p2, far):
        sel_ref[b, p2] = far
        cx = xyzs_ref[0, 0, far]
        cy = xyzs_ref[0, 1, far]
        cz = xyzs_ref[0, 2, far]
        xyzsel_ref[b, 0, p2] = cx
        xyzsel_ref[b, 1, p2] = cy
        xyzsel_ref[b, 2, p2] = cz
        return cx, cy, cz

    def region_body(r, p):
        member = (region == r) & valid
        take = take_ref[b, r]
        fps_b = ufps_ref[b, r] > 0
        steps = jnp.minimum(take, _T - p)

        def fps_path(p):
            score0 = jnp.where(member, jnp.float32(1e10), neg)

            def step(_, carry):
                p2, score = carry
                mx = jnp.max(score)
                far = jnp.min(jnp.where(score == mx, iota_i, jnp.int32(_K)))
                cx, cy, cz = emit(p2, far)
                dx = xs - cx
                dy = ys - cy
                dz = zs - cz
                d = (dx * dx + dy * dy) + dz * dz
                return (p2 + 1, jnp.minimum(score, d))

            return lax.fori_loop(0, steps, step, (p, score0))[0]

        def ord_path(p):
            score0 = jnp.where(member, -iota_f, neg)

            def step(_, carry):
                p2, score = carry
                mx = jnp.max(score)
                far = jnp.min(jnp.where(score == mx, iota_i, jnp.int32(_K)))
                emit(p2, far)
                return (p2 + 1, jnp.where(iota_i == far, neg, score))

            return lax.fori_loop(0, steps, step, (p, score0))[0]

        return lax.cond(fps_b, fps_path, ord_path, p)

    lax.fori_loop(0, _NREG, region_body, jnp.int32(0))


def _select(xyzT, xyzS, take, ufps):
    return pl.pallas_call(
        _select_body,
        grid=(_B,),
        in_specs=[
            pl.BlockSpec((1, 3, _R, _R), lambda b: (b, 0, 0, 0)),
            pl.BlockSpec((1, 3, _K), lambda b: (b, 0, 0),
                         memory_space=pltpu.SMEM),
            pl.BlockSpec(memory_space=pltpu.SMEM),
            pl.BlockSpec(memory_space=pltpu.SMEM),
        ],
        out_specs=[
            pl.BlockSpec(memory_space=pltpu.SMEM),
            pl.BlockSpec(memory_space=pltpu.SMEM),
        ],
        out_shape=[
            jax.ShapeDtypeStruct((_B, _T), jnp.int32),
            jax.ShapeDtypeStruct((_B, 3, _T), jnp.float32),
        ],
    )(xyzT, xyzS, take, ufps)


def _gather_rows(table, gidx):
    mesh = plsc.VectorSubcoreMesh(core_axis_name="c", subcore_axis_name="s")

    @functools.partial(
        pl.kernel,
        mesh=mesh,
        out_type=jax.ShapeDtypeStruct((_B * _T, _FEAT), jnp.float32),
        scratch_types=[
            pltpu.VMEM((_BPW,), jnp.int32),
            pltpu.VMEM((_BPW, _FEAT), jnp.float32),
            pltpu.SemaphoreType.DMA,
        ],
    )
    def k(table_hbm, idx_hbm, out_hbm, idx_v, rows_v, sem):
        wid = lax.axis_index("s") * _NC + lax.axis_index("c")
        base = wid * _BPW
        pltpu.sync_copy(idx_hbm.at[pl.ds(base, _BPW)], idx_v)
        pltpu.async_copy(table_hbm.at[idx_v], rows_v, sem).wait()
        pltpu.sync_copy(rows_v, out_hbm.at[pl.ds(base, _BPW)])

    return k(table, gidx)


def _proj_body(rows_ref, wa_ref, ba_ref, len_ref, o_ref):
    b = pl.program_id(0)
    rows = rows_ref[0]
    fo = lax.dot_general(wa_ref[...], rows, (((0,), (1,)), ((), ())),
                         preferred_element_type=jnp.float32)
    fo = fo + ba_ref[...]
    tpos = lax.broadcasted_iota(jnp.int32, (1, _T), 1)
    o_ref[0] = jnp.where(tpos < len_ref[b, 0], fo, 0.0)


def _project(rows, Wa, ba, sel_len):
    return pl.pallas_call(
        _proj_body,
        grid=(_B,),
        in_specs=[
            pl.BlockSpec((1, _T, _FEAT), lambda b: (b, 0, 0)),
            pl.BlockSpec((_FEAT, _TOKEN_DIM), lambda b: (0, 0)),
            pl.BlockSpec((_TOKEN_DIM, 1), lambda b: (0, 0)),
            pl.BlockSpec(memory_space=pltpu.SMEM),
        ],
        out_specs=pl.BlockSpec((1, _TOKEN_DIM, _T), lambda b: (b, 0, 0)),
        out_shape=jax.ShapeDtypeStruct((_B, _TOKEN_DIM, _T), jnp.float32),
    )(rows, Wa, ba.reshape(_TOKEN_DIM, 1), sel_len.reshape(_B, 1))


def kernel(xyz, features, W1, b1, W2, b2, Wa, ba):
    xyzS = jnp.transpose(xyz, (0, 2, 1))
    xyzT = xyzS.reshape(_B, 3, _R, _R)

    imp = _importance(features, W1, b1, W2, b2)
    impf = imp.reshape(_B, _R, _R)

    rimp, cnt = _region_stats(xyzT, impf)

    n_rows, take_rows, len_rows = [], [], []
    for b in range(_B):
        total = rimp[b].sum() + 1e-08
        n_b = jnp.round(rimp[b] / total * _T).astype(jnp.int32)
        take_b = jnp.minimum(cnt[b], n_b)
        n_rows.append(n_b)
        take_rows.append(take_b)
        len_rows.append(jnp.minimum(jnp.sum(take_b), _T).astype(jnp.int32))
    n_r = jnp.stack(n_rows)
    take = jnp.stack(take_rows)
    sel_len = jnp.stack(len_rows)
    ufps = (cnt > n_r).astype(jnp.int32)

    sel, xyzsel = _select(xyzT, xyzS, take, ufps)

    maskT = jnp.arange(_T)[None, :] < sel_len[:, None]
    xyz_out = jnp.where(maskT[:, :, None],
                        jnp.transpose(xyzsel, (0, 2, 1)), jnp.float32(0))
    sel_m = jnp.where(maskT, sel, 0)

    feat_rows = jnp.transpose(features, (0, 2, 1)).reshape(_B * _K, _FEAT)
    gidx = (sel_m + jnp.arange(_B, dtype=jnp.int32)[:, None] * _K)
    rows = _gather_rows(feat_rows, gidx.reshape(_B * _T))
    feat_out = _project(rows.reshape(_B, _T, _FEAT), Wa, ba, sel_len)
    return (xyz_out, feat_out)
